# Initial kernel scaffold; baseline (speedup 1.0000x reference)
import functools

import jax
import jax.numpy as jnp
from jax import lax
from jax.experimental import pallas as pl
from jax.experimental.pallas import tpu as pltpu
from jax.experimental.pallas import tpu_sc as plsc

N, E, D = 10000, 320000, 128
NC, NS, L = 2, 16, 16
NW = NC * NS
EPT = E // NW
CH = 80
NCHUNK = EPT // CH
RPT = N // NS
ZR = 125

_mesh = plsc.VectorSubcoreMesh(core_axis_name="c", subcore_axis_name="s")


@functools.partial(
    pl.kernel,
    out_type=jax.ShapeDtypeStruct((NW, N), jnp.float32),
    mesh=_mesh,
    scratch_types=[
        pltpu.VMEM((EPT,), jnp.int32),
        pltpu.VMEM((EPT,), jnp.float32),
        pltpu.VMEM((N,), jnp.float32),
    ],
)
def _deg_kernel(col_hbm, ew_hbm, out_hbm, colv, eww, degv):
    cid = lax.axis_index("c")
    sid = lax.axis_index("s")
    wid = sid * NC + cid

    def zero(i, _):
        degv[pl.ds(i * L, L)] = jnp.zeros((L,), jnp.float32)
        return 0
    lax.fori_loop(0, N // L, zero, 0)

    pltpu.sync_copy(col_hbm.at[wid], colv)
    pltpu.sync_copy(ew_hbm.at[wid], eww)

    def body(j, _):
        c = colv[pl.ds(j * L, L)]
        w = eww[pl.ds(j * L, L)]
        plsc.addupdate_scatter(degv, [c], w)
        return 0
    lax.fori_loop(0, EPT // L, body, 0)

    pltpu.sync_copy(degv, out_hbm.at[wid])


@functools.partial(
    pl.kernel,
    out_type=jax.ShapeDtypeStruct((NC, N, D), jnp.float32),
    mesh=_mesh,
    scratch_types=[
        pltpu.VMEM((EPT,), jnp.int32),
        pltpu.VMEM((NCHUNK, CH), jnp.int32),
        pltpu.VMEM((EPT,), jnp.float32),
        pltpu.VMEM((CH, D), jnp.float32),
        pltpu.VMEM((CH, D), jnp.float32),
        pltpu.VMEM((ZR, D), jnp.float32),
        pltpu.VMEM_SHARED((N, D), jnp.float32),
        pltpu.SemaphoreType.DMA,
        pltpu.SemaphoreType.DMA,
    ],
)
def _agg_kernel(row_hbm, col_hbm, ew_hbm, xws_hbm, out_hbm,
                rowv, colv, eww, bufa, bufb, zbuf, acc_sh, sema, semb):
    cid = lax.axis_index("c")
    sid = lax.axis_index("s")
    wid = sid * NC + cid

    def zfill(i, _):
        r = i // (D // L)
        k = i % (D // L)
        zbuf[r, pl.ds(k * L, L)] = jnp.zeros((L,), jnp.float32)
        return 0
    lax.fori_loop(0, ZR * (D // L), zfill, 0)
    for z in range(RPT // ZR):
        pltpu.sync_copy(zbuf, acc_sh.at[pl.ds(sid * RPT + z * ZR, ZR)])

    pltpu.sync_copy(row_hbm.at[wid], rowv)
    pltpu.sync_copy(col_hbm.at[wid], colv)
    pltpu.sync_copy(ew_hbm.at[wid], eww)
    plsc.subcore_barrier()

    def gather_start(j, buf, sem):
        pltpu.async_copy(xws_hbm.at[rowv.at[pl.ds(j * CH, CH)]], buf, sem)

    def gather_wait(j, buf, sem):
        pltpu.make_async_copy(xws_hbm.at[rowv.at[pl.ds(j * CH, CH)]], buf,
                              sem).wait()

    def scale_scatter(j, buf):
        jb = j * CH

        def per_edge(e, _):
            s = eww[jb + e]
            for k in range(D // L):
                buf[e, pl.ds(k * L, L)] = buf[e, pl.ds(k * L, L)] * s
            return 0
        lax.fori_loop(0, CH, per_edge, 0)
        pltpu.sync_copy(buf, acc_sh.at[colv.at[j]], add=True)

    gather_start(0, bufa, sema)

    def pair(i, _):
        jj = i * 2
        gather_start(jj + 1, bufb, semb)
        gather_wait(jj, bufa, sema)
        scale_scatter(jj, bufa)
        gather_start(jj + 2, bufa, sema)
        gather_wait(jj + 1, bufb, semb)
        scale_scatter(jj + 1, bufb)
        return 0
    lax.fori_loop(0, (NCHUNK - 1) // 2, pair, 0)

    gather_wait(NCHUNK - 1, bufa, sema)
    scale_scatter(NCHUNK - 1, bufa)

    plsc.subcore_barrier()
    pltpu.sync_copy(acc_sh.at[pl.ds(sid * RPT, RPT)],
                    out_hbm.at[cid, pl.ds(sid * RPT, RPT)])


def _k2_body(degp, emb, w1, dinv_ref, xws_ref):
    deg = jnp.sum(degp[...], axis=0) + 1.0
    dinv = lax.rsqrt(deg)
    dinv_ref[...] = dinv
    xw = jnp.dot(emb[...], w1[...], preferred_element_type=jnp.float32)
    xws_ref[...] = xw * dinv[:, None]


def _k4_body(accp, xws1, dinv, b1, gamma, beta, w2, h_ref, xws2_ref):
    a = accp[...]
    dv = dinv[...][:, None]
    pre = dv * (a[0] + a[1] + xws1[...]) + b1[...][None, :]
    mean = jnp.mean(pre, axis=0)
    var = jnp.mean((pre - mean[None, :]) ** 2, axis=0)
    hb = (pre - mean[None, :]) * lax.rsqrt(var + 1e-5)[None, :] \
        * gamma[...][None, :] + beta[...][None, :]
    h = jnp.where(hb > 0, hb, jnp.expm1(hb))
    h_ref[...] = h
    xw2 = jnp.dot(h, w2[...], preferred_element_type=jnp.float32)
    xws2_ref[...] = xw2 * dv


def _k6_body(accp, xws2, dinv, b2, emb, h, out_ref):
    a = accp[...]
    h2 = dinv[...][:, None] * (a[0] + a[1] + xws2[...]) + b2[...][None, :]
    out_ref[...] = (emb[...] + h[...] + h2) * (1.0 / 3.0)


def kernel(emb, edge_index, edge_weight, W1, b1, gamma, beta, W2, b2):
    row2 = edge_index[0].reshape(NW, EPT)
    col2 = edge_index[1].reshape(NW, EPT)
    col3 = edge_index[1].reshape(NW, NCHUNK, CH)
    ew2 = edge_weight.reshape(NW, EPT)

    degp = _deg_kernel(col2, ew2)

    dinv, xws1 = pl.pallas_call(
        _k2_body,
        out_shape=(jax.ShapeDtypeStruct((N,), jnp.float32),
                   jax.ShapeDtypeStruct((N, D), jnp.float32)),
    )(degp, emb, W1)

    acc1 = _agg_kernel(row2, col3, ew2, xws1)

    h, xws2 = pl.pallas_call(
        _k4_body,
        out_shape=(jax.ShapeDtypeStruct((N, D), jnp.float32),
                   jax.ShapeDtypeStruct((N, D), jnp.float32)),
    )(acc1, xws1, dinv, b1, gamma, beta, W2)

    acc2 = _agg_kernel(row2, col3, ew2, xws2)

    out = pl.pallas_call(
        _k6_body,
        out_shape=jax.ShapeDtypeStruct((N, D), jnp.float32),
    )(acc2, xws2, dinv, b2, emb, h)
    return out


# speedup vs baseline: 18.0936x; 18.0936x over previous
import functools

import jax
import jax.numpy as jnp
from jax import lax
from jax.experimental import pallas as pl
from jax.experimental.pallas import tpu as pltpu
from jax.experimental.pallas import tpu_sc as plsc

N, E, D = 10000, 320000, 128
NC, NS, L = 2, 16, 16
NW = NC * NS
D2 = D // NC
EPT = E // NW
EPT2 = E // NS
CH = 80
NCHUNK = EPT2 // CH
SRT = 624
TAIL = N - NS * SRT
ZR = 208

_mesh = plsc.VectorSubcoreMesh(core_axis_name="c", subcore_axis_name="s")
_sc_params = pltpu.CompilerParams(needs_layout_passes=False,
                                  use_tc_tiling_on_sc=False)


@functools.partial(
    pl.kernel,
    out_type=jax.ShapeDtypeStruct((NW * N,), jnp.float32),
    mesh=_mesh,
    compiler_params=_sc_params,
    scratch_types=[
        pltpu.VMEM((EPT,), jnp.int32),
        pltpu.VMEM((EPT,), jnp.float32),
        pltpu.VMEM((N,), jnp.float32),
    ],
)
def _deg_kernel(col_hbm, ew_hbm, out_hbm, colv, eww, degv):
    cid = lax.axis_index("c")
    sid = lax.axis_index("s")
    wid = sid * NC + cid

    def zero(i, _):
        degv[pl.ds(i * L, L)] = jnp.zeros((L,), jnp.float32)
        return 0
    lax.fori_loop(0, N // L, zero, 0)

    pltpu.sync_copy(col_hbm.at[pl.ds(wid * EPT, EPT)], colv)
    pltpu.sync_copy(ew_hbm.at[pl.ds(wid * EPT, EPT)], eww)

    def body(j, _):
        c = colv[pl.ds(j * L, L)]
        w = eww[pl.ds(j * L, L)]
        plsc.addupdate_scatter(degv, [c], w)
        return 0
    lax.fori_loop(0, EPT // L, body, 0)

    pltpu.sync_copy(degv, out_hbm.at[pl.ds(wid * N, N)])


@functools.partial(
    pl.kernel,
    out_type=jax.ShapeDtypeStruct((NC, N, D2), jnp.float32),
    mesh=_mesh,
    compiler_params=_sc_params,
    scratch_types=[
        pltpu.VMEM((EPT2,), jnp.int32),
        pltpu.VMEM((NCHUNK, CH), jnp.int32),
        pltpu.VMEM((EPT2,), jnp.float32),
        pltpu.VMEM((CH, D2), jnp.float32),
        pltpu.VMEM((CH, D2), jnp.float32),
        pltpu.VMEM((ZR, D2), jnp.float32),
        pltpu.VMEM_SHARED((N, D2), jnp.float32),
        pltpu.SemaphoreType.DMA,
        pltpu.SemaphoreType.DMA,
    ],
)
def _agg_kernel(row_hbm, col_hbm, ew_hbm, xws_hbm, out_hbm,
                rowv, colv, eww, bufa, bufb, zbuf, acc_sh, sema, semb):
    cid = lax.axis_index("c")
    sid = lax.axis_index("s")

    def zfill(i, _):
        r = i // (D2 // L)
        k = i % (D2 // L)
        zbuf[r, pl.ds(k * L, L)] = jnp.zeros((L,), jnp.float32)
        return 0
    lax.fori_loop(0, ZR * (D2 // L), zfill, 0)
    for z in range(SRT // ZR):
        pltpu.sync_copy(zbuf, acc_sh.at[pl.ds(sid * SRT + z * ZR, ZR)])

    @pl.when(sid == 0)
    def _():
        pltpu.sync_copy(zbuf.at[pl.ds(0, TAIL)],
                        acc_sh.at[pl.ds(NS * SRT, TAIL)])

    pltpu.sync_copy(row_hbm.at[pl.ds(sid * EPT2, EPT2)], rowv)
    pltpu.sync_copy(col_hbm.at[sid], colv)
    pltpu.sync_copy(ew_hbm.at[pl.ds(sid * EPT2, EPT2)], eww)

    def to_half(i, _):
        v = rowv[pl.ds(i * L, L)]
        rowv[pl.ds(i * L, L)] = v * 2 + cid
        return 0
    lax.fori_loop(0, EPT2 // L, to_half, 0)

    plsc.subcore_barrier()

    def gather_start(j, buf, sem):
        pltpu.async_copy(xws_hbm.at[rowv.at[pl.ds(j * CH, CH)]], buf, sem)

    def gather_wait(j, buf, sem):
        pltpu.make_async_copy(xws_hbm.at[rowv.at[pl.ds(j * CH, CH)]], buf,
                              sem).wait()

    def scale_scatter(j, buf):
        jb = j * CH

        def per_group(g, _):
            w16 = eww[pl.ds(jb + g * L, L)]
            for e in range(L):
                s = w16[e]
                r = g * L + e
                for k in range(D2 // L):
                    buf[r, pl.ds(k * L, L)] = buf[r, pl.ds(k * L, L)] * s
            return 0
        lax.fori_loop(0, CH // L, per_group, 0)
        pltpu.sync_copy(buf, acc_sh.at[colv.at[j]], add=True)

    gather_start(0, bufa, sema)

    def pair(i, _):
        jj = i * 2
        gather_start(jj + 1, bufb, semb)
        gather_wait(jj, bufa, sema)
        scale_scatter(jj, bufa)

        @pl.when(jj + 2 < NCHUNK)
        def _():
            gather_start(jj + 2, bufa, sema)
        gather_wait(jj + 1, bufb, semb)
        scale_scatter(jj + 1, bufb)
        return 0
    lax.fori_loop(0, NCHUNK // 2, pair, 0)

    plsc.subcore_barrier()
    pltpu.sync_copy(acc_sh.at[pl.ds(sid * SRT, SRT)],
                    out_hbm.at[cid, pl.ds(sid * SRT, SRT)])

    @pl.when(sid == 0)
    def _():
        pltpu.sync_copy(acc_sh.at[pl.ds(NS * SRT, TAIL)],
                        out_hbm.at[cid, pl.ds(NS * SRT, TAIL)])


def _k2_body(degp, emb, w1, dinv_ref, xws_ref):
    deg = jnp.sum(degp[...], axis=0) + 1.0
    dinv = lax.rsqrt(deg)
    dinv_ref[...] = dinv
    xw = jnp.dot(emb[...], w1[...], preferred_element_type=jnp.float32)
    xws_ref[...] = xw * dinv[:, None]


def _k4_body(accp, xws1, dinv, b1, gamma, beta, w2, h_ref, xws2_ref):
    a = accp[...]
    acc = jnp.concatenate([a[0], a[1]], axis=1)
    dv = dinv[...][:, None]
    pre = dv * (acc + xws1[...]) + b1[...][None, :]
    mean = jnp.mean(pre, axis=0)
    var = jnp.mean((pre - mean[None, :]) ** 2, axis=0)
    hb = (pre - mean[None, :]) * lax.rsqrt(var + 1e-5)[None, :] \
        * gamma[...][None, :] + beta[...][None, :]
    h = jnp.where(hb > 0, hb, jnp.exp(jnp.minimum(hb, 0.0)) - 1.0)
    h_ref[...] = h
    xw2 = jnp.dot(h, w2[...], preferred_element_type=jnp.float32)
    xws2_ref[...] = xw2 * dv


def _k6_body(accp, xws2, dinv, b2, emb, h, out_ref):
    a = accp[...]
    acc = jnp.concatenate([a[0], a[1]], axis=1)
    h2 = dinv[...][:, None] * (acc + xws2[...]) + b2[...][None, :]
    out_ref[...] = (emb[...] + h[...] + h2) * (1.0 / 3.0)


def kernel(emb, edge_index, edge_weight, W1, b1, gamma, beta, W2, b2):
    row1 = edge_index[0]
    col1 = edge_index[1]
    col3 = edge_index[1].reshape(NS, NCHUNK, CH)

    degp = _deg_kernel(col1, edge_weight).reshape(NW, N)

    dinv, xws1 = pl.pallas_call(
        _k2_body,
        out_shape=(jax.ShapeDtypeStruct((N,), jnp.float32),
                   jax.ShapeDtypeStruct((N, D), jnp.float32)),
    )(degp, emb, W1)

    acc1 = _agg_kernel(row1, col3, edge_weight, xws1.reshape(NC * N, D2))

    h, xws2 = pl.pallas_call(
        _k4_body,
        out_shape=(jax.ShapeDtypeStruct((N, D), jnp.float32),
                   jax.ShapeDtypeStruct((N, D), jnp.float32)),
    )(acc1, xws1, dinv, b1, gamma, beta, W2)

    acc2 = _agg_kernel(row1, col3, edge_weight, xws2.reshape(NC * N, D2))

    out = pl.pallas_call(
        _k6_body,
        out_shape=jax.ShapeDtypeStruct((N, D), jnp.float32),
    )(acc2, xws2, dinv, b2, emb, h)
    return out


# speedup vs baseline: 20.4869x; 1.1323x over previous
import functools

import jax
import jax.numpy as jnp
from jax import lax
from jax.experimental import pallas as pl
from jax.experimental.pallas import tpu as pltpu
from jax.experimental.pallas import tpu_sc as plsc

N, E, D = 10000, 320000, 128
NC, NS, L = 2, 16, 16
NW = NC * NS
D2 = D // NC
EPT = E // NW
EPT2 = E // NS
CH = 80
NCHUNK = EPT2 // CH
SRT = 624
TAIL = N - NS * SRT
ZR = 208

_mesh = plsc.VectorSubcoreMesh(core_axis_name="c", subcore_axis_name="s")
_sc_params = pltpu.CompilerParams(needs_layout_passes=False,
                                  use_tc_tiling_on_sc=False)


@functools.partial(
    pl.kernel,
    out_type=jax.ShapeDtypeStruct((NW * N,), jnp.float32),
    mesh=_mesh,
    compiler_params=_sc_params,
    scratch_types=[
        pltpu.VMEM((EPT,), jnp.int32),
        pltpu.VMEM((EPT,), jnp.float32),
        pltpu.VMEM((N,), jnp.float32),
    ],
)
def _deg_kernel(col_hbm, ew_hbm, out_hbm, colv, eww, degv):
    cid = lax.axis_index("c")
    sid = lax.axis_index("s")
    wid = sid * NC + cid

    def zero(i, _):
        degv[pl.ds(i * L, L)] = jnp.zeros((L,), jnp.float32)
        return 0
    lax.fori_loop(0, N // L, zero, 0)

    pltpu.sync_copy(col_hbm.at[pl.ds(wid * EPT, EPT)], colv)
    pltpu.sync_copy(ew_hbm.at[pl.ds(wid * EPT, EPT)], eww)

    def body(j, _):
        c = colv[pl.ds(j * L, L)]
        w = eww[pl.ds(j * L, L)]
        plsc.addupdate_scatter(degv, [c], w)
        return 0
    lax.fori_loop(0, EPT // L, body, 0)

    pltpu.sync_copy(degv, out_hbm.at[pl.ds(wid * N, N)])


@functools.partial(
    pl.kernel,
    out_type=jax.ShapeDtypeStruct((NC, N, D2), jnp.float32),
    mesh=_mesh,
    compiler_params=_sc_params,
    scratch_types=[
        pltpu.VMEM((EPT2,), jnp.int32),
        pltpu.VMEM((NCHUNK, CH), jnp.int32),
        pltpu.VMEM((EPT2,), jnp.float32),
        pltpu.VMEM((CH, D2), jnp.float32),
        pltpu.VMEM((CH, D2), jnp.float32),
        pltpu.VMEM((CH, D2), jnp.float32),
        pltpu.VMEM((ZR, D2), jnp.float32),
        pltpu.VMEM_SHARED((N, D2), jnp.float32),
        pltpu.SemaphoreType.DMA,
        pltpu.SemaphoreType.DMA,
        pltpu.SemaphoreType.DMA,
        pltpu.SemaphoreType.DMA,
        pltpu.SemaphoreType.DMA,
        pltpu.SemaphoreType.DMA,
    ],
)
def _agg_kernel(row_hbm, col_hbm, ew_hbm, xws_hbm, out_hbm,
                rowv, colv, eww, buf0, buf1, buf2, zbuf, acc_sh,
                gs0, gs1, gs2, ss0, ss1, ss2):
    cid = lax.axis_index("c")
    sid = lax.axis_index("s")

    def zfill(i, _):
        r = i // (D2 // L)
        k = i % (D2 // L)
        zbuf[r, pl.ds(k * L, L)] = jnp.zeros((L,), jnp.float32)
        return 0
    lax.fori_loop(0, ZR * (D2 // L), zfill, 0)
    for z in range(SRT // ZR):
        pltpu.sync_copy(zbuf, acc_sh.at[pl.ds(sid * SRT + z * ZR, ZR)])

    @pl.when(sid == 0)
    def _():
        pltpu.sync_copy(zbuf.at[pl.ds(0, TAIL)],
                        acc_sh.at[pl.ds(NS * SRT, TAIL)])

    pltpu.sync_copy(row_hbm.at[pl.ds(sid * EPT2, EPT2)], rowv)
    pltpu.sync_copy(col_hbm.at[sid], colv)
    pltpu.sync_copy(ew_hbm.at[pl.ds(sid * EPT2, EPT2)], eww)

    def to_half(i, _):
        v = rowv[pl.ds(i * L, L)]
        rowv[pl.ds(i * L, L)] = v * 2 + cid
        return 0
    lax.fori_loop(0, EPT2 // L, to_half, 0)

    plsc.subcore_barrier()

    def gather_start(j, buf, sem):
        pltpu.async_copy(xws_hbm.at[rowv.at[pl.ds(j * CH, CH)]], buf, sem)

    def gather_wait(j, buf, sem):
        pltpu.make_async_copy(xws_hbm.at[rowv.at[pl.ds(j * CH, CH)]], buf,
                              sem).wait()

    def scale(j, buf):
        jb = j * CH

        def per_group(g, _):
            w16 = eww[pl.ds(jb + g * L, L)]
            for e in range(L):
                s = w16[e]
                r = g * L + e
                for k in range(D2 // L):
                    buf[r, pl.ds(k * L, L)] = buf[r, pl.ds(k * L, L)] * s
            return 0
        lax.fori_loop(0, CH // L, per_group, 0)

    def scatter_start(j, buf, sem):
        pltpu.async_copy(buf, acc_sh.at[colv.at[j]], sem, add=True)

    def scatter_wait(j, buf, sem):
        pltpu.make_async_copy(buf, acc_sh.at[colv.at[j]], sem).wait()

    bufs = (buf0, buf1, buf2)
    gsems = (gs0, gs1, gs2)
    ssems = (ss0, ss1, ss2)
    NB = 3

    gather_start(0, buf0, gs0)
    gather_start(1, buf1, gs1)

    def step(j, _):
        m = j % NB
        for o in range(NB):
            p = (o + 2) % NB

            @pl.when(m == o)
            def _(o=o, p=p):
                gather_wait(j, bufs[o], gsems[o])
                scale(j, bufs[o])
                scatter_start(j, bufs[o], ssems[o])

                @pl.when(j + 2 < NCHUNK)
                def _():
                    @pl.when(j >= 1)
                    def _():
                        scatter_wait(j - 1, bufs[p], ssems[p])
                    gather_start(j + 2, bufs[p], gsems[p])
        return 0
    lax.fori_loop(0, NCHUNK, step, 0)

    scatter_wait(NCHUNK - 2, bufs[(NCHUNK - 2) % NB], ssems[(NCHUNK - 2) % NB])
    scatter_wait(NCHUNK - 1, bufs[(NCHUNK - 1) % NB], ssems[(NCHUNK - 1) % NB])

    plsc.subcore_barrier()
    pltpu.sync_copy(acc_sh.at[pl.ds(sid * SRT, SRT)],
                    out_hbm.at[cid, pl.ds(sid * SRT, SRT)])

    @pl.when(sid == 0)
    def _():
        pltpu.sync_copy(acc_sh.at[pl.ds(NS * SRT, TAIL)],
                        out_hbm.at[cid, pl.ds(NS * SRT, TAIL)])


def _k2_body(degp, emb, w1, dinv_ref, xws_ref):
    deg = jnp.sum(degp[...], axis=0) + 1.0
    dinv = lax.rsqrt(deg)
    dinv_ref[...] = dinv
    xw = jnp.dot(emb[...], w1[...], preferred_element_type=jnp.float32)
    xws_ref[...] = xw * dinv[:, None]


def _k4_body(accp, xws1, dinv, b1, gamma, beta, w2, h_ref, xws2_ref):
    a = accp[...]
    acc = jnp.concatenate([a[0], a[1]], axis=1)
    dv = dinv[...][:, None]
    pre = dv * (acc + xws1[...]) + b1[...][None, :]
    mean = jnp.mean(pre, axis=0)
    var = jnp.mean((pre - mean[None, :]) ** 2, axis=0)
    hb = (pre - mean[None, :]) * lax.rsqrt(var + 1e-5)[None, :] \
        * gamma[...][None, :] + beta[...][None, :]
    h = jnp.where(hb > 0, hb, jnp.exp(jnp.minimum(hb, 0.0)) - 1.0)
    h_ref[...] = h
    xw2 = jnp.dot(h, w2[...], preferred_element_type=jnp.float32)
    xws2_ref[...] = xw2 * dv


def _k6_body(accp, xws2, dinv, b2, emb, h, out_ref):
    a = accp[...]
    acc = jnp.concatenate([a[0], a[1]], axis=1)
    h2 = dinv[...][:, None] * (acc + xws2[...]) + b2[...][None, :]
    out_ref[...] = (emb[...] + h[...] + h2) * (1.0 / 3.0)


def kernel(emb, edge_index, edge_weight, W1, b1, gamma, beta, W2, b2):
    row1 = edge_index[0]
    col1 = edge_index[1]
    col3 = edge_index[1].reshape(NS, NCHUNK, CH)

    degp = _deg_kernel(col1, edge_weight).reshape(NW, N)

    dinv, xws1 = pl.pallas_call(
        _k2_body,
        out_shape=(jax.ShapeDtypeStruct((N,), jnp.float32),
                   jax.ShapeDtypeStruct((N, D), jnp.float32)),
    )(degp, emb, W1)

    acc1 = _agg_kernel(row1, col3, edge_weight, xws1.reshape(NC * N, D2))

    h, xws2 = pl.pallas_call(
        _k4_body,
        out_shape=(jax.ShapeDtypeStruct((N, D), jnp.float32),
                   jax.ShapeDtypeStruct((N, D), jnp.float32)),
    )(acc1, xws1, dinv, b1, gamma, beta, W2)

    acc2 = _agg_kernel(row1, col3, edge_weight, xws2.reshape(NC * N, D2))

    out = pl.pallas_call(
        _k6_body,
        out_shape=jax.ShapeDtypeStruct((N, D), jnp.float32),
    )(acc2, xws2, dinv, b2, emb, h)
    return out


# speedup vs baseline: 20.5116x; 1.0012x over previous
import functools

import jax
import jax.numpy as jnp
from jax import lax
from jax.experimental import pallas as pl
from jax.experimental.pallas import tpu as pltpu
from jax.experimental.pallas import tpu_sc as plsc

N, E, D = 10000, 320000, 128
NC, NS, L = 2, 16, 16
NW = NC * NS
D2 = D // NC
EPT = E // NW
EPT2 = E // NS
CH = 80
NCHUNK = EPT2 // CH
SRT = 624
TAIL = N - NS * SRT
ZR = 208

_mesh = plsc.VectorSubcoreMesh(core_axis_name="c", subcore_axis_name="s")
_sc_params = pltpu.CompilerParams(needs_layout_passes=False,
                                  use_tc_tiling_on_sc=False)


@functools.partial(
    pl.kernel,
    out_type=jax.ShapeDtypeStruct((NW * N,), jnp.float32),
    mesh=_mesh,
    compiler_params=_sc_params,
    scratch_types=[
        pltpu.VMEM((EPT,), jnp.int32),
        pltpu.VMEM((EPT,), jnp.float32),
        pltpu.VMEM((N,), jnp.float32),
    ],
)
def _deg_kernel(col_hbm, ew_hbm, out_hbm, colv, eww, degv):
    cid = lax.axis_index("c")
    sid = lax.axis_index("s")
    wid = sid * NC + cid

    def zero(i, _):
        degv[pl.ds(i * L, L)] = jnp.zeros((L,), jnp.float32)
        return 0
    lax.fori_loop(0, N // L, zero, 0)

    pltpu.sync_copy(col_hbm.at[pl.ds(wid * EPT, EPT)], colv)
    pltpu.sync_copy(ew_hbm.at[pl.ds(wid * EPT, EPT)], eww)

    def body(j, _):
        c = colv[pl.ds(j * L, L)]
        w = eww[pl.ds(j * L, L)]
        plsc.addupdate_scatter(degv, [c], w)
        return 0
    lax.fori_loop(0, EPT // L, body, 0)

    pltpu.sync_copy(degv, out_hbm.at[pl.ds(wid * N, N)])


@functools.partial(
    pl.kernel,
    out_type=jax.ShapeDtypeStruct((NC, N, D2), jnp.float32),
    mesh=_mesh,
    compiler_params=_sc_params,
    scratch_types=[
        pltpu.VMEM((EPT2,), jnp.int32),
        pltpu.VMEM((NCHUNK, CH), jnp.int32),
        pltpu.VMEM((EPT2,), jnp.float32),
        pltpu.VMEM((CH, D2), jnp.float32),
        pltpu.VMEM((CH, D2), jnp.float32),
        pltpu.VMEM((CH, D2), jnp.float32),
        pltpu.VMEM((ZR, D2), jnp.float32),
        pltpu.VMEM_SHARED((N, D2), jnp.float32),
        pltpu.SemaphoreType.DMA,
        pltpu.SemaphoreType.DMA,
        pltpu.SemaphoreType.DMA,
        pltpu.SemaphoreType.DMA,
        pltpu.SemaphoreType.DMA,
        pltpu.SemaphoreType.DMA,
    ],
)
def _agg_kernel(row_hbm, col_hbm, ew_hbm, xws_hbm, out_hbm,
                rowv, colv, eww, buf0, buf1, buf2, zbuf, acc_sh,
                gs0, gs1, gs2, ss0, ss1, ss2):
    cid = lax.axis_index("c")
    sid = lax.axis_index("s")

    def zfill(i, _):
        r = i // (D2 // L)
        k = i % (D2 // L)
        zbuf[r, pl.ds(k * L, L)] = jnp.zeros((L,), jnp.float32)
        return 0
    lax.fori_loop(0, ZR * (D2 // L), zfill, 0)
    for z in range(SRT // ZR):
        pltpu.sync_copy(zbuf, acc_sh.at[pl.ds(sid * SRT + z * ZR, ZR)])

    @pl.when(sid == 0)
    def _():
        pltpu.sync_copy(zbuf.at[pl.ds(0, TAIL)],
                        acc_sh.at[pl.ds(NS * SRT, TAIL)])

    pltpu.sync_copy(row_hbm.at[pl.ds(sid * EPT2, EPT2)], rowv)
    pltpu.sync_copy(col_hbm.at[sid], colv)
    pltpu.sync_copy(ew_hbm.at[pl.ds(sid * EPT2, EPT2)], eww)

    def to_half(i, _):
        v = rowv[pl.ds(i * L, L)]
        rowv[pl.ds(i * L, L)] = v * 2 + cid
        return 0
    lax.fori_loop(0, EPT2 // L, to_half, 0)

    plsc.subcore_barrier()

    def gather_start(j, buf, sem):
        pltpu.async_copy(xws_hbm.at[rowv.at[pl.ds(j * CH, CH)]], buf, sem)

    def gather_wait(j, buf, sem):
        pltpu.make_async_copy(xws_hbm.at[rowv.at[pl.ds(j * CH, CH)]], buf,
                              sem).wait()

    def scale(j, buf):
        jb = j * CH

        def per_group(g, _):
            w16 = eww[pl.ds(jb + g * L, L)]
            for e in range(L):
                s = w16[e]
                r = g * L + e
                for k in range(D2 // L):
                    buf[r, pl.ds(k * L, L)] = buf[r, pl.ds(k * L, L)] * s
            return 0
        lax.fori_loop(0, CH // L, per_group, 0)

    def scatter_start(j, buf, sem):
        pltpu.async_copy(buf, acc_sh.at[colv.at[j]], sem, add=True)

    def scatter_wait(j, buf, sem):
        pltpu.make_async_copy(buf, acc_sh.at[colv.at[j]], sem).wait()

    bufs = (buf0, buf1, buf2)
    gsems = (gs0, gs1, gs2)
    ssems = (ss0, ss1, ss2)
    NB = 3

    gather_start(0, buf0, gs0)
    gather_start(1, buf1, gs1)

    def step(j, _):
        m = j % NB
        for o in range(NB):
            p = (o + 2) % NB

            @pl.when(m == o)
            def _(o=o, p=p):
                gather_wait(j, bufs[o], gsems[o])
                scale(j, bufs[o])
                scatter_start(j, bufs[o], ssems[o])

                @pl.when(j + 2 < NCHUNK)
                def _():
                    @pl.when(j >= 1)
                    def _():
                        scatter_wait(j - 1, bufs[p], ssems[p])
                    gather_start(j + 2, bufs[p], gsems[p])
        return 0
    lax.fori_loop(0, NCHUNK, step, 0)

    for c in (NCHUNK - 3, NCHUNK - 2, NCHUNK - 1):
        scatter_wait(c, bufs[c % NB], ssems[c % NB])

    plsc.subcore_barrier()
    pltpu.sync_copy(acc_sh.at[pl.ds(sid * SRT, SRT)],
                    out_hbm.at[cid, pl.ds(sid * SRT, SRT)])

    @pl.when(sid == 0)
    def _():
        pltpu.sync_copy(acc_sh.at[pl.ds(NS * SRT, TAIL)],
                        out_hbm.at[cid, pl.ds(NS * SRT, TAIL)])


def _k2_body(degp, emb, w1, dinv_ref, xws_ref):
    deg = jnp.sum(degp[...], axis=0) + 1.0
    dinv = lax.rsqrt(deg)
    dinv_ref[...] = dinv
    xw = jnp.dot(emb[...], w1[...], preferred_element_type=jnp.float32)
    xws_ref[...] = xw * dinv[:, None]


def _k4_body(accp, xws1, dinv, b1, gamma, beta, w2, h_ref, xws2_ref):
    a = accp[...]
    acc = jnp.concatenate([a[0], a[1]], axis=1)
    dv = dinv[...][:, None]
    pre = dv * (acc + xws1[...]) + b1[...][None, :]
    mean = jnp.mean(pre, axis=0)
    var = jnp.mean((pre - mean[None, :]) ** 2, axis=0)
    hb = (pre - mean[None, :]) * lax.rsqrt(var + 1e-5)[None, :] \
        * gamma[...][None, :] + beta[...][None, :]
    h = jnp.where(hb > 0, hb, jnp.exp(jnp.minimum(hb, 0.0)) - 1.0)
    h_ref[...] = h
    xw2 = jnp.dot(h, w2[...], preferred_element_type=jnp.float32)
    xws2_ref[...] = xw2 * dv


def _k6_body(accp, xws2, dinv, b2, emb, h, out_ref):
    a = accp[...]
    acc = jnp.concatenate([a[0], a[1]], axis=1)
    h2 = dinv[...][:, None] * (acc + xws2[...]) + b2[...][None, :]
    out_ref[...] = (emb[...] + h[...] + h2) * (1.0 / 3.0)


def kernel(emb, edge_index, edge_weight, W1, b1, gamma, beta, W2, b2):
    row1 = edge_index[0]
    col1 = edge_index[1]
    col3 = edge_index[1].reshape(NS, NCHUNK, CH)

    degp = _deg_kernel(col1, edge_weight).reshape(NW, N)

    dinv, xws1 = pl.pallas_call(
        _k2_body,
        out_shape=(jax.ShapeDtypeStruct((N,), jnp.float32),
                   jax.ShapeDtypeStruct((N, D), jnp.float32)),
    )(degp, emb, W1)

    acc1 = _agg_kernel(row1, col3, edge_weight, xws1.reshape(NC * N, D2))

    h, xws2 = pl.pallas_call(
        _k4_body,
        out_shape=(jax.ShapeDtypeStruct((N, D), jnp.float32),
                   jax.ShapeDtypeStruct((N, D), jnp.float32)),
    )(acc1, xws1, dinv, b1, gamma, beta, W2)

    acc2 = _agg_kernel(row1, col3, edge_weight, xws2.reshape(NC * N, D2))

    out = pl.pallas_call(
        _k6_body,
        out_shape=jax.ShapeDtypeStruct((N, D), jnp.float32),
    )(acc2, xws2, dinv, b2, emb, h)
    return out


# speedup vs baseline: 22.7183x; 1.1076x over previous
import functools

import jax
import jax.numpy as jnp
from jax import lax
from jax.experimental import pallas as pl
from jax.experimental.pallas import tpu as pltpu
from jax.experimental.pallas import tpu_sc as plsc

N, E, D = 10000, 320000, 128
NC, NS, L = 2, 16, 16
NW = NC * NS
D2 = D // NC
EPT = E // NW
EPT2 = E // NS
CH = 80
NCHUNK = EPT2 // CH
SRT = 624
TAIL = N - NS * SRT
ZR = 208

_mesh = plsc.VectorSubcoreMesh(core_axis_name="c", subcore_axis_name="s")
_sc_params = pltpu.CompilerParams(needs_layout_passes=False,
                                  use_tc_tiling_on_sc=False)


@functools.partial(
    pl.kernel,
    out_type=jax.ShapeDtypeStruct((NW * N,), jnp.float32),
    mesh=_mesh,
    compiler_params=_sc_params,
    scratch_types=[
        pltpu.VMEM((EPT,), jnp.int32),
        pltpu.VMEM((EPT,), jnp.float32),
        pltpu.VMEM((N,), jnp.float32),
    ],
)
def _deg_kernel(col_hbm, ew_hbm, out_hbm, colv, eww, degv):
    cid = lax.axis_index("c")
    sid = lax.axis_index("s")
    wid = sid * NC + cid

    def zero(i, _):
        degv[pl.ds(i * L, L)] = jnp.zeros((L,), jnp.float32)
        return 0
    lax.fori_loop(0, N // L, zero, 0)

    pltpu.sync_copy(col_hbm.at[pl.ds(wid * EPT, EPT)], colv)
    pltpu.sync_copy(ew_hbm.at[pl.ds(wid * EPT, EPT)], eww)

    def body(j, _):
        c = colv[pl.ds(j * L, L)]
        w = eww[pl.ds(j * L, L)]
        plsc.addupdate_scatter(degv, [c], w)
        return 0
    lax.fori_loop(0, EPT // L, body, 0)

    pltpu.sync_copy(degv, out_hbm.at[pl.ds(wid * N, N)])


@functools.partial(
    pl.kernel,
    out_type=jax.ShapeDtypeStruct((NC, N, D2), jnp.float32),
    mesh=_mesh,
    compiler_params=_sc_params,
    scratch_types=[
        pltpu.VMEM((EPT2,), jnp.int32),
        pltpu.VMEM((NCHUNK, CH), jnp.int32),
        pltpu.VMEM((EPT2,), jnp.float32),
        pltpu.VMEM((CH, D2), jnp.float32),
        pltpu.VMEM((CH, D2), jnp.float32),
        pltpu.VMEM((CH, D2), jnp.float32),
        pltpu.VMEM((ZR, D2), jnp.float32),
        pltpu.VMEM_SHARED((N, D2), jnp.float32),
        pltpu.SemaphoreType.DMA,
        pltpu.SemaphoreType.DMA,
        pltpu.SemaphoreType.DMA,
        pltpu.SemaphoreType.DMA,
        pltpu.SemaphoreType.DMA,
        pltpu.SemaphoreType.DMA,
    ],
)
def _agg_kernel(row_hbm, col_hbm, ew_hbm, xws_hbm, out_hbm,
                rowv, colv, eww, buf0, buf1, buf2, zbuf, acc_sh,
                gs0, gs1, gs2, ss0, ss1, ss2):
    cid = lax.axis_index("c")
    sid = lax.axis_index("s")

    def zfill(i, _):
        r = i // (D2 // L)
        k = i % (D2 // L)
        zbuf[r, pl.ds(k * L, L)] = jnp.zeros((L,), jnp.float32)
        return 0
    lax.fori_loop(0, ZR * (D2 // L), zfill, 0)
    for z in range(SRT // ZR):
        pltpu.sync_copy(zbuf, acc_sh.at[pl.ds(sid * SRT + z * ZR, ZR)])

    @pl.when(sid == 0)
    def _():
        pltpu.sync_copy(zbuf.at[pl.ds(0, TAIL)],
                        acc_sh.at[pl.ds(NS * SRT, TAIL)])

    pltpu.sync_copy(row_hbm.at[pl.ds(sid * EPT2, EPT2)], rowv)
    pltpu.sync_copy(col_hbm.at[sid], colv)
    pltpu.sync_copy(ew_hbm.at[pl.ds(sid * EPT2, EPT2)], eww)

    def to_half(i, _):
        v = rowv[pl.ds(i * L, L)]
        rowv[pl.ds(i * L, L)] = v * 2 + cid
        return 0
    lax.fori_loop(0, EPT2 // L, to_half, 0)

    plsc.subcore_barrier()

    def gather_start(j, buf, sem):
        pltpu.async_copy(xws_hbm.at[rowv.at[pl.ds(j * CH, CH)]], buf, sem)

    def gather_wait(j, buf, sem):
        pltpu.make_async_copy(xws_hbm.at[rowv.at[pl.ds(j * CH, CH)]], buf,
                              sem).wait()

    def scale(j, buf):
        jb = j * CH

        @plsc.parallel_loop(0, CH // L, unroll=2)
        def per_group(g):
            w16 = eww[pl.ds(jb + g * L, L)]
            for e in range(L):
                s = w16[e]
                r = g * L + e
                for k in range(D2 // L):
                    buf[r, pl.ds(k * L, L)] = buf[r, pl.ds(k * L, L)] * s

    def scatter_start(j, buf, sem):
        pltpu.async_copy(buf, acc_sh.at[colv.at[j]], sem, add=True)

    def scatter_wait(j, buf, sem):
        pltpu.make_async_copy(buf, acc_sh.at[colv.at[j]], sem).wait()

    bufs = (buf0, buf1, buf2)
    gsems = (gs0, gs1, gs2)
    ssems = (ss0, ss1, ss2)
    NB = 3

    gather_start(0, buf0, gs0)
    gather_start(1, buf1, gs1)

    def step(j, _):
        m = j % NB
        for o in range(NB):
            p = (o + 2) % NB

            @pl.when(m == o)
            def _(o=o, p=p):
                gather_wait(j, bufs[o], gsems[o])
                scale(j, bufs[o])
                scatter_start(j, bufs[o], ssems[o])

                @pl.when(j + 2 < NCHUNK)
                def _():
                    @pl.when(j >= 1)
                    def _():
                        scatter_wait(j - 1, bufs[p], ssems[p])
                    gather_start(j + 2, bufs[p], gsems[p])
        return 0
    lax.fori_loop(0, NCHUNK, step, 0)

    for c in (NCHUNK - 3, NCHUNK - 2, NCHUNK - 1):
        scatter_wait(c, bufs[c % NB], ssems[c % NB])

    plsc.subcore_barrier()
    pltpu.sync_copy(acc_sh.at[pl.ds(sid * SRT, SRT)],
                    out_hbm.at[cid, pl.ds(sid * SRT, SRT)])

    @pl.when(sid == 0)
    def _():
        pltpu.sync_copy(acc_sh.at[pl.ds(NS * SRT, TAIL)],
                        out_hbm.at[cid, pl.ds(NS * SRT, TAIL)])


def _k2_body(degp, emb, w1, dinv_ref, xws_ref):
    deg = jnp.sum(degp[...], axis=0) + 1.0
    dinv = lax.rsqrt(deg)
    dinv_ref[...] = dinv
    xw = jnp.dot(emb[...], w1[...], preferred_element_type=jnp.float32)
    xws_ref[...] = xw * dinv[:, None]


def _k4_body(accp, xws1, dinv, b1, gamma, beta, w2, h_ref, xws2_ref):
    a = accp[...]
    acc = jnp.concatenate([a[0], a[1]], axis=1)
    dv = dinv[...][:, None]
    pre = dv * (acc + xws1[...]) + b1[...][None, :]
    mean = jnp.mean(pre, axis=0)
    var = jnp.mean((pre - mean[None, :]) ** 2, axis=0)
    hb = (pre - mean[None, :]) * lax.rsqrt(var + 1e-5)[None, :] \
        * gamma[...][None, :] + beta[...][None, :]
    h = jnp.where(hb > 0, hb, jnp.exp(jnp.minimum(hb, 0.0)) - 1.0)
    h_ref[...] = h
    xw2 = jnp.dot(h, w2[...], preferred_element_type=jnp.float32)
    xws2_ref[...] = xw2 * dv


def _k6_body(accp, xws2, dinv, b2, emb, h, out_ref):
    a = accp[...]
    acc = jnp.concatenate([a[0], a[1]], axis=1)
    h2 = dinv[...][:, None] * (acc + xws2[...]) + b2[...][None, :]
    out_ref[...] = (emb[...] + h[...] + h2) * (1.0 / 3.0)


def kernel(emb, edge_index, edge_weight, W1, b1, gamma, beta, W2, b2):
    row1 = edge_index[0]
    col1 = edge_index[1]
    col3 = edge_index[1].reshape(NS, NCHUNK, CH)

    degp = _deg_kernel(col1, edge_weight).reshape(NW, N)

    dinv, xws1 = pl.pallas_call(
        _k2_body,
        out_shape=(jax.ShapeDtypeStruct((N,), jnp.float32),
                   jax.ShapeDtypeStruct((N, D), jnp.float32)),
    )(degp, emb, W1)

    acc1 = _agg_kernel(row1, col3, edge_weight, xws1.reshape(NC * N, D2))

    h, xws2 = pl.pallas_call(
        _k4_body,
        out_shape=(jax.ShapeDtypeStruct((N, D), jnp.float32),
                   jax.ShapeDtypeStruct((N, D), jnp.float32)),
    )(acc1, xws1, dinv, b1, gamma, beta, W2)

    acc2 = _agg_kernel(row1, col3, edge_weight, xws2.reshape(NC * N, D2))

    out = pl.pallas_call(
        _k6_body,
        out_shape=jax.ShapeDtypeStruct((N, D), jnp.float32),
    )(acc2, xws2, dinv, b2, emb, h)
    return out


# speedup vs baseline: 22.7631x; 1.0020x over previous
import functools

import jax
import jax.numpy as jnp
from jax import lax
from jax.experimental import pallas as pl
from jax.experimental.pallas import tpu as pltpu
from jax.experimental.pallas import tpu_sc as plsc

N, E, D = 10000, 320000, 128
NC, NS, L = 2, 16, 16
NW = NC * NS
D2 = D // NC
EPT = E // NW
EPT2 = E // NS
CH = 80
NCHUNK = EPT2 // CH
SRT = 624
TAIL = N - NS * SRT
ZR = 208

_mesh = plsc.VectorSubcoreMesh(core_axis_name="c", subcore_axis_name="s")
_sc_params = pltpu.CompilerParams(needs_layout_passes=False,
                                  use_tc_tiling_on_sc=False)


@functools.partial(
    pl.kernel,
    out_type=jax.ShapeDtypeStruct((NW * N,), jnp.float32),
    mesh=_mesh,
    compiler_params=_sc_params,
    scratch_types=[
        pltpu.VMEM((EPT,), jnp.int32),
        pltpu.VMEM((EPT,), jnp.float32),
        pltpu.VMEM((N,), jnp.float32),
    ],
)
def _deg_kernel(col_hbm, ew_hbm, out_hbm, colv, eww, degv):
    cid = lax.axis_index("c")
    sid = lax.axis_index("s")
    wid = sid * NC + cid

    def zero(i, _):
        degv[pl.ds(i * L, L)] = jnp.zeros((L,), jnp.float32)
        return 0
    lax.fori_loop(0, N // L, zero, 0)

    pltpu.sync_copy(col_hbm.at[pl.ds(wid * EPT, EPT)], colv)
    pltpu.sync_copy(ew_hbm.at[pl.ds(wid * EPT, EPT)], eww)

    def body(j, _):
        c = colv[pl.ds(j * L, L)]
        w = eww[pl.ds(j * L, L)]
        plsc.addupdate_scatter(degv, [c], w)
        return 0
    lax.fori_loop(0, EPT // L, body, 0)

    pltpu.sync_copy(degv, out_hbm.at[pl.ds(wid * N, N)])


@functools.partial(
    pl.kernel,
    out_type=jax.ShapeDtypeStruct((NC, N, D2), jnp.float32),
    mesh=_mesh,
    compiler_params=_sc_params,
    scratch_types=[
        pltpu.VMEM((EPT2,), jnp.int32),
        pltpu.VMEM((NCHUNK, CH), jnp.int32),
        pltpu.VMEM((EPT2,), jnp.float32),
        pltpu.VMEM((CH, D2), jnp.float32),
        pltpu.VMEM((CH, D2), jnp.float32),
        pltpu.VMEM((CH, D2), jnp.float32),
        pltpu.VMEM((ZR, D2), jnp.float32),
        pltpu.VMEM_SHARED((N, D2), jnp.float32),
        pltpu.SemaphoreType.DMA,
        pltpu.SemaphoreType.DMA,
        pltpu.SemaphoreType.DMA,
        pltpu.SemaphoreType.DMA,
        pltpu.SemaphoreType.DMA,
        pltpu.SemaphoreType.DMA,
    ],
)
def _agg_kernel(row_hbm, col_hbm, ew_hbm, xws_hbm, out_hbm,
                rowv, colv, eww, buf0, buf1, buf2, zbuf, acc_sh,
                gs0, gs1, gs2, ss0, ss1, ss2):
    cid = lax.axis_index("c")
    sid = lax.axis_index("s")

    def zfill(i, _):
        r = i // (D2 // L)
        k = i % (D2 // L)
        zbuf[r, pl.ds(k * L, L)] = jnp.zeros((L,), jnp.float32)
        return 0
    lax.fori_loop(0, ZR * (D2 // L), zfill, 0)
    for z in range(SRT // ZR):
        pltpu.sync_copy(zbuf, acc_sh.at[pl.ds(sid * SRT + z * ZR, ZR)])

    @pl.when(sid == 0)
    def _():
        pltpu.sync_copy(zbuf.at[pl.ds(0, TAIL)],
                        acc_sh.at[pl.ds(NS * SRT, TAIL)])

    pltpu.sync_copy(row_hbm.at[pl.ds(sid * EPT2, EPT2)], rowv)
    pltpu.sync_copy(col_hbm.at[sid], colv)
    pltpu.sync_copy(ew_hbm.at[pl.ds(sid * EPT2, EPT2)], eww)

    def to_half(i, _):
        v = rowv[pl.ds(i * L, L)]
        rowv[pl.ds(i * L, L)] = v * 2 + cid
        return 0
    lax.fori_loop(0, EPT2 // L, to_half, 0)

    plsc.subcore_barrier()

    def gather_start(j, buf, sem):
        pltpu.async_copy(xws_hbm.at[rowv.at[pl.ds(j * CH, CH)]], buf, sem)

    def gather_wait(j, buf, sem):
        pltpu.make_async_copy(xws_hbm.at[rowv.at[pl.ds(j * CH, CH)]], buf,
                              sem).wait()

    def scale(j, buf):
        jb = j * CH

        @plsc.parallel_loop(0, CH // L, unroll=5)
        def per_group(g):
            w16 = eww[pl.ds(jb + g * L, L)]
            for e in range(L):
                s = w16[e]
                r = g * L + e
                for k in range(D2 // L):
                    buf[r, pl.ds(k * L, L)] = buf[r, pl.ds(k * L, L)] * s

    def scatter_start(j, buf, sem):
        pltpu.async_copy(buf, acc_sh.at[colv.at[j]], sem, add=True)

    def scatter_wait(j, buf, sem):
        pltpu.make_async_copy(buf, acc_sh.at[colv.at[j]], sem).wait()

    bufs = (buf0, buf1, buf2)
    gsems = (gs0, gs1, gs2)
    ssems = (ss0, ss1, ss2)
    NB = 3

    gather_start(0, buf0, gs0)
    gather_start(1, buf1, gs1)

    def step(j, _):
        m = j % NB
        for o in range(NB):
            p = (o + 2) % NB

            @pl.when(m == o)
            def _(o=o, p=p):
                gather_wait(j, bufs[o], gsems[o])
                scale(j, bufs[o])
                scatter_start(j, bufs[o], ssems[o])

                @pl.when(j + 2 < NCHUNK)
                def _():
                    @pl.when(j >= 1)
                    def _():
                        scatter_wait(j - 1, bufs[p], ssems[p])
                    gather_start(j + 2, bufs[p], gsems[p])
        return 0
    lax.fori_loop(0, NCHUNK, step, 0)

    for c in (NCHUNK - 3, NCHUNK - 2, NCHUNK - 1):
        scatter_wait(c, bufs[c % NB], ssems[c % NB])

    plsc.subcore_barrier()
    pltpu.sync_copy(acc_sh.at[pl.ds(sid * SRT, SRT)],
                    out_hbm.at[cid, pl.ds(sid * SRT, SRT)])

    @pl.when(sid == 0)
    def _():
        pltpu.sync_copy(acc_sh.at[pl.ds(NS * SRT, TAIL)],
                        out_hbm.at[cid, pl.ds(NS * SRT, TAIL)])


def _k2_body(degp, emb, w1, dinv_ref, xws_ref):
    deg = jnp.sum(degp[...], axis=0) + 1.0
    dinv = lax.rsqrt(deg)
    dinv_ref[...] = dinv
    xw = jnp.dot(emb[...], w1[...], preferred_element_type=jnp.float32)
    xws_ref[...] = xw * dinv[:, None]


def _k4_body(accp, xws1, dinv, b1, gamma, beta, w2, h_ref, xws2_ref):
    a = accp[...]
    acc = jnp.concatenate([a[0], a[1]], axis=1)
    dv = dinv[...][:, None]
    pre = dv * (acc + xws1[...]) + b1[...][None, :]
    mean = jnp.mean(pre, axis=0)
    var = jnp.mean((pre - mean[None, :]) ** 2, axis=0)
    hb = (pre - mean[None, :]) * lax.rsqrt(var + 1e-5)[None, :] \
        * gamma[...][None, :] + beta[...][None, :]
    h = jnp.where(hb > 0, hb, jnp.exp(jnp.minimum(hb, 0.0)) - 1.0)
    h_ref[...] = h
    xw2 = jnp.dot(h, w2[...], preferred_element_type=jnp.float32)
    xws2_ref[...] = xw2 * dv


def _k6_body(accp, xws2, dinv, b2, emb, h, out_ref):
    a = accp[...]
    acc = jnp.concatenate([a[0], a[1]], axis=1)
    h2 = dinv[...][:, None] * (acc + xws2[...]) + b2[...][None, :]
    out_ref[...] = (emb[...] + h[...] + h2) * (1.0 / 3.0)


def kernel(emb, edge_index, edge_weight, W1, b1, gamma, beta, W2, b2):
    row1 = edge_index[0]
    col1 = edge_index[1]
    col3 = edge_index[1].reshape(NS, NCHUNK, CH)

    degp = _deg_kernel(col1, edge_weight).reshape(NW, N)

    dinv, xws1 = pl.pallas_call(
        _k2_body,
        out_shape=(jax.ShapeDtypeStruct((N,), jnp.float32),
                   jax.ShapeDtypeStruct((N, D), jnp.float32)),
    )(degp, emb, W1)

    acc1 = _agg_kernel(row1, col3, edge_weight, xws1.reshape(NC * N, D2))

    h, xws2 = pl.pallas_call(
        _k4_body,
        out_shape=(jax.ShapeDtypeStruct((N, D), jnp.float32),
                   jax.ShapeDtypeStruct((N, D), jnp.float32)),
    )(acc1, xws1, dinv, b1, gamma, beta, W2)

    acc2 = _agg_kernel(row1, col3, edge_weight, xws2.reshape(NC * N, D2))

    out = pl.pallas_call(
        _k6_body,
        out_shape=jax.ShapeDtypeStruct((N, D), jnp.float32),
    )(acc2, xws2, dinv, b2, emb, h)
    return out
